# baseline (speedup 1.0000x reference)
import functools

import jax
import jax.numpy as jnp
from jax import lax
from jax.experimental import pallas as pl
from jax.experimental.pallas import tpu as pltpu
from jax.experimental.pallas import tpu_sc as plsc

N_NODES = 10000
D_H = 128
N_SC = 2
N_TILES = 16
N_PAD = 10240
ROWS_PER_TILE = N_PAD // N_TILES
ZCHUNK = 128
CW = 16
ROW_BLOCK = 2000


def _make_sc_agg(n_edges_per_core: int, chunk: int, with_counts: bool):
    e_per_tile = n_edges_per_core // N_TILES
    n_chunks = e_per_tile // chunk
    assert e_per_tile % chunk == 0 and chunk % 8 == 0 and chunk <= 128
    assert n_chunks % 2 == 1
    assert ROWS_PER_TILE % chunk == 0
    if with_counts:
        assert chunk % 16 == 0

    mesh = plsc.VectorSubcoreMesh(core_axis_name="c", subcore_axis_name="s")
    feat_ty = jax.ShapeDtypeStruct((N_SC, N_PAD, D_H), jnp.float32)
    if with_counts:
        out_type = [feat_ty,
                    jax.ShapeDtypeStruct((N_SC * N_PAD,), jnp.float32),
                    jax.ShapeDtypeStruct((N_SC * N_TILES * N_PAD,),
                                         jnp.float32)]
    else:
        out_type = feat_ty
    scratch = [
        pltpu.VMEM((chunk,), jnp.int32),
        pltpu.VMEM((chunk,), jnp.int32),
        pltpu.VMEM((chunk,), jnp.int32),
        pltpu.VMEM((chunk,), jnp.int32),
        pltpu.VMEM((chunk, D_H), jnp.float32),
        pltpu.VMEM((chunk, D_H), jnp.float32),
        pltpu.VMEM_SHARED((N_PAD, D_H), jnp.float32),
        pltpu.SemaphoreType.DMA,
        pltpu.SemaphoreType.DMA,
        pltpu.SemaphoreType.DMA,
        pltpu.SemaphoreType.DMA,
    ]
    if with_counts:
        scratch += [
            pltpu.VMEM((N_PAD,), jnp.float32),
            pltpu.VMEM((N_TILES, ROWS_PER_TILE), jnp.float32),
            pltpu.VMEM((ROWS_PER_TILE,), jnp.float32),
        ]

    @functools.partial(
        pl.kernel, out_type=out_type, mesh=mesh, scratch_types=scratch,
        compiler_params=pltpu.CompilerParams(use_tc_tiling_on_sc=False,
                                             needs_layout_passes=False))
    def sc_agg(table0, table1, src0, dst0, src1, dst1, z_rows, z_hist, out,
               *rest):
        if with_counts:
            cnt_out, hist_raw, sidx0, sidx1, didx0, didx1, rows0, rows1, \
                accum, gs0, gs1, ss0, ss1, hist, hbuf, cnt_buf = rest
        else:
            (sidx0, sidx1, didx0, didx1, rows0, rows1, accum,
             gs0, gs1, ss0, ss1) = rest
        c = lax.axis_index("c")
        s = lax.axis_index("s")

        pltpu.sync_copy(z_rows,
                        accum.at[pl.ds(s * ROWS_PER_TILE, ROWS_PER_TILE)])
        if with_counts:
            pltpu.sync_copy(z_hist, hist)
        plsc.subcore_barrier()

        ones16 = jnp.ones((16,), jnp.float32)
        ebase = s * e_per_tile

        def run_edges(table, src, dst):
            def iload(j, sb, db):
                base = ebase + j * chunk
                pltpu.sync_copy(src.at[pl.ds(base, chunk)], sb)
                pltpu.sync_copy(dst.at[pl.ds(base, chunk)], db)

            def gstart(sb, buf, sem):
                pltpu.async_copy(table.at[sb], buf, sem)

            def gwait(buf, sem):
                pltpu.make_async_copy(table.at[sidx0], buf, sem).wait()

            def sstart(buf, db, sem):
                pltpu.async_copy(buf, accum.at[db], sem, add=True)
                if with_counts:
                    for t in range(chunk // 16):
                        dv = db[pl.ds(t * 16, 16)]
                        plsc.addupdate_scatter(hist, [dv], ones16)

            def swait(buf, db, sem):
                pltpu.make_async_copy(buf, accum.at[db], sem).wait()

            iload(0, sidx0, didx0)
            gstart(sidx0, rows0, gs0)

            def body(j2, carry):
                c0 = 2 * j2
                iload(c0 + 1, sidx1, didx1)
                gstart(sidx1, rows1, gs1)
                gwait(rows0, gs0)
                sstart(rows0, didx0, ss0)
                gwait(rows1, gs1)
                swait(rows0, didx0, ss0)
                sstart(rows1, didx1, ss1)
                iload(c0 + 2, sidx0, didx0)
                gstart(sidx0, rows0, gs0)
                swait(rows1, didx1, ss1)
                return carry
            lax.fori_loop(0, (n_chunks - 1) // 2, body, 0)
            gwait(rows0, gs0)
            sstart(rows0, didx0, ss0)
            swait(rows0, didx0, ss0)

        @pl.when(c == 0)
        def _():
            run_edges(table0, src0, dst0)

        @pl.when(c == 1)
        def _():
            run_edges(table1, src1, dst1)

        if with_counts:
            pltpu.sync_copy(
                hist, hist_raw.at[pl.ds((c * N_TILES + s) * N_PAD, N_PAD)])
        plsc.subcore_barrier()

        if with_counts:
            for t in range(N_TILES):
                pltpu.sync_copy(
                    hist_raw.at[pl.ds((c * N_TILES + t) * N_PAD
                                      + s * ROWS_PER_TILE, ROWS_PER_TILE)],
                    hbuf.at[t])

            def fold_body(k, carry):
                acc = hbuf[0, pl.ds(k * 16, 16)]
                for t in range(1, N_TILES):
                    acc = acc + hbuf[t, pl.ds(k * 16, 16)]
                cnt_buf[pl.ds(k * 16, 16)] = acc
                return carry
            lax.fori_loop(0, ROWS_PER_TILE // 16, fold_body, 0)
            pltpu.sync_copy(
                cnt_buf,
                cnt_out.at[pl.ds(c * N_PAD + s * ROWS_PER_TILE,
                                 ROWS_PER_TILE)])

        r0 = s * ROWS_PER_TILE
        pltpu.sync_copy(accum.at[pl.ds(r0, ROWS_PER_TILE)],
                        out.at[c, pl.ds(r0, ROWS_PER_TILE)])

    return sc_agg


_sc_agg_stage1 = _make_sc_agg(160000, 80, with_counts=True)
_sc_agg_stage2 = _make_sc_agg(80000, 40, with_counts=False)


def _row_spec(width):
    return pl.BlockSpec((ROW_BLOCK, width), lambda i: (i, 0))


def _full_spec(shape):
    return pl.BlockSpec(shape, lambda i: tuple(0 for _ in shape))


def _tc1_body(xu, xi, Wiu, biu, Wii, bii, Wsu, Wsi, Wdu, Wdi, bdi, bdu,
              Au, Ai, Di, Du):
    f32 = jnp.float32
    hu = jnp.dot(xu[...], Wiu[...], preferred_element_type=f32) + biu[...]
    hi = jnp.dot(xi[...], Wii[...], preferred_element_type=f32) + bii[...]
    Au[...] = jnp.dot(hu, Wsu[...], preferred_element_type=f32)
    Ai[...] = jnp.dot(hi, Wsi[...], preferred_element_type=f32)
    Di[...] = jnp.dot(hi, Wdu[...], preferred_element_type=f32) + bdi[...]
    Du[...] = jnp.dot(hu, Wdi[...], preferred_element_type=f32) + bdu[...]


def _tc1(xu, xi, Wiu, biu, Wii, bii, Wsu, Wsi, Wdu, Wdi, bdi, bdu):
    w = _full_spec((D_H, D_H))
    b = _full_spec((1, D_H))
    r = _row_spec(D_H)
    return pl.pallas_call(
        _tc1_body,
        grid=(N_NODES // ROW_BLOCK,),
        in_specs=[r, r, w, b, w, b, w, w, w, w, b, b],
        out_specs=[r, r, r, r],
        out_shape=[jax.ShapeDtypeStruct((N_NODES, D_H), jnp.float32)] * 4,
    )(xu, xi, Wiu, biu, Wii, bii, Wsu, Wsi, Wdu, Wdi, bdi, bdu)


def _leaky(x):
    return jnp.where(x > 0, x, 0.01 * x)


def _tc2_body(Si, ci, Di, Su, cu, Du, Wsi1, Wdi1, bdu1, Ai1, Du1):
    f32 = jnp.float32
    hi1 = _leaky(Si[...] / jnp.maximum(ci[...], 1.0) + Di[...])
    hu1 = _leaky(Su[...] / jnp.maximum(cu[...], 1.0) + Du[...])
    Ai1[...] = jnp.dot(hi1, Wsi1[...], preferred_element_type=f32)
    Du1[...] = jnp.dot(hu1, Wdi1[...], preferred_element_type=f32) + bdu1[...]


def _tc2(Si, ci, Di, Su, cu, Du, Wsi1, Wdi1, bdu1):
    w = _full_spec((D_H, D_H))
    b = _full_spec((1, D_H))
    r = _row_spec(D_H)
    c = _row_spec(1)
    return pl.pallas_call(
        _tc2_body,
        grid=(N_NODES // ROW_BLOCK,),
        in_specs=[r, c, r, r, c, r, w, w, b],
        out_specs=[r, r],
        out_shape=[jax.ShapeDtypeStruct((N_NODES, D_H), jnp.float32)] * 2,
    )(Si, ci, Di, Su, cu, Du, Wsi1, Wdi1, bdu1)


def _tc3_body(Sp0, Sp1, cu, Du1, Wout, bout, out):
    h = (Sp0[...] + Sp1[...]) / jnp.maximum(cu[...], 1.0) + Du1[...]
    out[...] = jnp.dot(h, Wout[...], preferred_element_type=jnp.float32) \
        + bout[...]


def _tc3(Sp0, Sp1, cu, Du1, Wout, bout):
    n_out = Wout.shape[1]
    return pl.pallas_call(
        _tc3_body,
        grid=(N_NODES // ROW_BLOCK,),
        in_specs=[_row_spec(D_H), _row_spec(D_H), _row_spec(1), _row_spec(D_H),
                  _full_spec((D_H, n_out)), _full_spec((1, n_out))],
        out_specs=_row_spec(n_out),
        out_shape=jax.ShapeDtypeStruct((N_NODES, n_out), jnp.float32),
    )(Sp0, Sp1, cu, Du1, Wout, bout)


def kernel(x_user, x_item, edge_index_u2i, edge_index_i2u,
           W_in_user, b_in_user, W_in_item, b_in_item,
           W_src_u2i, b_src_u2i, W_dst_u2i, b_dst_u2i,
           W_src_i2u, b_src_i2u, W_dst_i2u, b_dst_i2u,
           W_out, b_out):
    i32 = jnp.int32
    src_u2i = edge_index_u2i[0].astype(i32)
    dst_u2i = edge_index_u2i[1].astype(i32)
    src_i2u = edge_index_i2u[0].astype(i32)
    dst_i2u = edge_index_i2u[1].astype(i32)
    half = src_i2u.shape[0] // 2

    z_rows = jnp.zeros((ROWS_PER_TILE, D_H), jnp.float32)
    z_hist = jnp.zeros((N_PAD,), jnp.float32)

    bDi0 = (b_src_u2i[0] + b_dst_u2i[0])[None]
    bDu0 = (b_src_i2u[0] + b_dst_i2u[0])[None]
    bDu1 = (b_src_i2u[1] + b_dst_i2u[1])[None]

    Au0, Ai0, Di0, Du0 = _tc1(
        x_user, x_item, W_in_user, b_in_user[None], W_in_item, b_in_item[None],
        W_src_u2i[0], W_src_i2u[0], W_dst_u2i[0], W_dst_i2u[0], bDi0, bDu0)

    S1, cnt_flat, _hr = _sc_agg_stage1(Au0, Ai0, src_u2i, dst_u2i, src_i2u,
                                       dst_i2u, z_rows, z_hist)
    cnt = cnt_flat.reshape(N_SC, N_PAD)
    cnt_i = cnt[0, :N_NODES][:, None]
    cnt_u = cnt[1, :N_NODES][:, None]

    Ai1, Du1 = _tc2(S1[0, :N_NODES], cnt_i, Di0, S1[1, :N_NODES], cnt_u, Du0,
                    W_src_i2u[1], W_dst_i2u[1], bDu1)

    S2 = _sc_agg_stage2(Ai1, Ai1, src_i2u[:half], dst_i2u[:half],
                        src_i2u[half:], dst_i2u[half:], z_rows, z_hist)

    return _tc3(S2[0, :N_NODES], S2[1, :N_NODES], cnt_u, Du1,
                W_out, b_out[None])


# speedup vs baseline: 1.4457x; 1.4457x over previous
import functools

import jax
import jax.numpy as jnp
from jax import lax
from jax.experimental import pallas as pl
from jax.experimental.pallas import tpu as pltpu
from jax.experimental.pallas import tpu_sc as plsc

N_NODES = 10000
D_H = 128
N_SC = 2
N_TILES = 16
N_PAD = 10240
ROWS_PER_TILE = N_PAD // N_TILES
ZCHUNK = 128
CW = 16
ROW_BLOCK = 2000


def _make_sc_agg(n_edges_per_core: int, chunk: int, with_counts: bool):
    e_per_tile = n_edges_per_core // N_TILES
    n_chunks = e_per_tile // chunk
    assert e_per_tile % chunk == 0 and chunk % 8 == 0 and chunk <= 128
    assert ROWS_PER_TILE % chunk == 0
    if with_counts:
        assert chunk % 16 == 0

    mesh = plsc.VectorSubcoreMesh(core_axis_name="c", subcore_axis_name="s")
    feat_ty = jax.ShapeDtypeStruct((N_SC, N_PAD, D_H), jnp.float32)
    if with_counts:
        out_type = [feat_ty,
                    jax.ShapeDtypeStruct((N_SC * N_PAD,), jnp.float32),
                    jax.ShapeDtypeStruct((N_SC * N_TILES * N_PAD,),
                                         jnp.float32)]
    else:
        out_type = feat_ty
    NB = 25
    assert n_chunks % NB == 0
    nb = n_chunks // NB
    bwords = NB * chunk
    scratch = [
        pltpu.VMEM((bwords,), jnp.int32),
        pltpu.VMEM((bwords,), jnp.int32),
        pltpu.VMEM((bwords,), jnp.int32),
        pltpu.VMEM((bwords,), jnp.int32),
        pltpu.VMEM((chunk, D_H), jnp.float32),
        pltpu.VMEM((chunk, D_H), jnp.float32),
        pltpu.VMEM_SHARED((N_PAD, D_H), jnp.float32),
        pltpu.SemaphoreType.DMA,
        pltpu.SemaphoreType.DMA,
        pltpu.SemaphoreType.DMA,
    ]
    FW = 320
    if with_counts:
        scratch += [
            pltpu.VMEM((N_PAD,), jnp.float32),
            pltpu.VMEM((N_TILES, FW), jnp.float32),
            pltpu.VMEM((ROWS_PER_TILE,), jnp.float32),
        ]

    @functools.partial(
        pl.kernel, out_type=out_type, mesh=mesh, scratch_types=scratch,
        compiler_params=pltpu.CompilerParams(use_tc_tiling_on_sc=False,
                                             needs_layout_passes=False))
    def sc_agg(table0, table1, src0, dst0, src1, dst1, z_rows, z_hist, out,
               *rest):
        if with_counts:
            cnt_out, hist_raw, sb0, sb1, db0, db1, rows0, rows1, \
                accum, gs0, gs1, bsem, hist, hbuf, cnt_buf = rest
        else:
            (sb0, sb1, db0, db1, rows0, rows1, accum, gs0, gs1, bsem) = rest
        c = lax.axis_index("c")
        s = lax.axis_index("s")

        pltpu.sync_copy(z_rows,
                        accum.at[pl.ds(s * ROWS_PER_TILE, ROWS_PER_TILE)])
        if with_counts:
            pltpu.sync_copy(z_hist, hist)
        plsc.subcore_barrier()

        ones16 = jnp.ones((16,), jnp.float32)
        ebase = s * e_per_tile

        def run_edges(table, src, dst):
            bufs = ((sb0, db0), (sb1, db1))

            def bstart(b, bufpair):
                base = ebase + b * bwords
                pltpu.async_copy(src.at[pl.ds(base, bwords)], bufpair[0], bsem)
                pltpu.async_copy(dst.at[pl.ds(base, bwords)], bufpair[1], bsem)

            def bwait(bufpair):
                pltpu.make_async_copy(
                    src.at[pl.ds(0, bwords)], bufpair[0], bsem).wait()
                pltpu.make_async_copy(
                    dst.at[pl.ds(0, bwords)], bufpair[1], bsem).wait()

            def gstart(sb, k, buf, sem):
                pltpu.async_copy(
                    table.at[sb.at[pl.ds(k * chunk, chunk)]], buf, sem)

            def gwait(buf, sem):
                pltpu.make_async_copy(
                    table.at[sb0.at[pl.ds(0, chunk)]], buf, sem).wait()

            def consume(buf, db, k):
                pltpu.sync_copy(
                    buf, accum.at[db.at[pl.ds(k * chunk, chunk)]], add=True)
                if with_counts:
                    for t in range(chunk // 16):
                        dv = db[pl.ds(k * chunk + t * 16, 16)]
                        plsc.addupdate_scatter(hist, [dv], ones16)

            bstart(0, bufs[0])
            bwait(bufs[0])
            for b in range(nb):
                sb, db = bufs[b % 2]
                if b + 1 < nb:
                    bstart(b + 1, bufs[(b + 1) % 2])
                gstart(sb, 0, rows0, gs0)

                def body(k2, carry, sb=sb, db=db):
                    k0 = 2 * k2
                    gstart(sb, k0 + 1, rows1, gs1)
                    gwait(rows0, gs0)
                    consume(rows0, db, k0)
                    gstart(sb, k0 + 2, rows0, gs0)
                    gwait(rows1, gs1)
                    consume(rows1, db, k0 + 1)
                    return carry
                lax.fori_loop(0, (NB - 1) // 2, body, 0)
                gwait(rows0, gs0)
                consume(rows0, db, NB - 1)
                if b + 1 < nb:
                    bwait(bufs[(b + 1) % 2])

        @pl.when(c == 0)
        def _():
            run_edges(table0, src0, dst0)

        @pl.when(c == 1)
        def _():
            run_edges(table1, src1, dst1)

        if with_counts:
            pltpu.sync_copy(
                hist, hist_raw.at[pl.ds((c * N_TILES + s) * N_PAD, N_PAD)])
        plsc.subcore_barrier()

        if with_counts:
            for half in range(ROWS_PER_TILE // FW):
                cb = s * ROWS_PER_TILE + half * FW
                for t in range(N_TILES):
                    pltpu.sync_copy(
                        hist_raw.at[pl.ds((c * N_TILES + t) * N_PAD + cb,
                                          FW)],
                        hbuf.at[t])

                def fold_body(k, carry, half=half):
                    acc = hbuf[0, pl.ds(k * 16, 16)]
                    for t in range(1, N_TILES):
                        acc = acc + hbuf[t, pl.ds(k * 16, 16)]
                    cnt_buf[pl.ds(half * FW + k * 16, 16)] = acc
                    return carry
                lax.fori_loop(0, FW // 16, fold_body, 0)
            pltpu.sync_copy(
                cnt_buf,
                cnt_out.at[pl.ds(c * N_PAD + s * ROWS_PER_TILE,
                                 ROWS_PER_TILE)])

        r0 = s * ROWS_PER_TILE
        pltpu.sync_copy(accum.at[pl.ds(r0, ROWS_PER_TILE)],
                        out.at[c, pl.ds(r0, ROWS_PER_TILE)])

    return sc_agg


_sc_agg_stage1 = _make_sc_agg(160000, 80, with_counts=True)
_sc_agg_stage2 = _make_sc_agg(80000, 40, with_counts=False)


def _row_spec(width):
    return pl.BlockSpec((ROW_BLOCK, width), lambda i: (i, 0))


def _full_spec(shape):
    return pl.BlockSpec(shape, lambda i: tuple(0 for _ in shape))


def _tc1_body(xu, xi, Wiu, biu, Wii, bii, Wsu, Wsi, Wdu, Wdi, bdi, bdu,
              Au, Ai, Di, Du):
    f32 = jnp.float32
    hu = jnp.dot(xu[...], Wiu[...], preferred_element_type=f32) + biu[...]
    hi = jnp.dot(xi[...], Wii[...], preferred_element_type=f32) + bii[...]
    Au[...] = jnp.dot(hu, Wsu[...], preferred_element_type=f32)
    Ai[...] = jnp.dot(hi, Wsi[...], preferred_element_type=f32)
    Di[...] = jnp.dot(hi, Wdu[...], preferred_element_type=f32) + bdi[...]
    Du[...] = jnp.dot(hu, Wdi[...], preferred_element_type=f32) + bdu[...]


def _tc1(xu, xi, Wiu, biu, Wii, bii, Wsu, Wsi, Wdu, Wdi, bdi, bdu):
    w = _full_spec((D_H, D_H))
    b = _full_spec((1, D_H))
    r = _row_spec(D_H)
    return pl.pallas_call(
        _tc1_body,
        grid=(N_NODES // ROW_BLOCK,),
        in_specs=[r, r, w, b, w, b, w, w, w, w, b, b],
        out_specs=[r, r, r, r],
        out_shape=[jax.ShapeDtypeStruct((N_NODES, D_H), jnp.float32)] * 4,
    )(xu, xi, Wiu, biu, Wii, bii, Wsu, Wsi, Wdu, Wdi, bdi, bdu)


def _leaky(x):
    return jnp.where(x > 0, x, 0.01 * x)


def _tc2_body(Si, ci, Di, Su, cu, Du, Wsi1, Wdi1, bdu1, Ai1, Du1):
    f32 = jnp.float32
    hi1 = _leaky(Si[...] / jnp.maximum(ci[...], 1.0) + Di[...])
    hu1 = _leaky(Su[...] / jnp.maximum(cu[...], 1.0) + Du[...])
    Ai1[...] = jnp.dot(hi1, Wsi1[...], preferred_element_type=f32)
    Du1[...] = jnp.dot(hu1, Wdi1[...], preferred_element_type=f32) + bdu1[...]


def _tc2(Si, ci, Di, Su, cu, Du, Wsi1, Wdi1, bdu1):
    w = _full_spec((D_H, D_H))
    b = _full_spec((1, D_H))
    r = _row_spec(D_H)
    c = _row_spec(1)
    return pl.pallas_call(
        _tc2_body,
        grid=(N_NODES // ROW_BLOCK,),
        in_specs=[r, c, r, r, c, r, w, w, b],
        out_specs=[r, r],
        out_shape=[jax.ShapeDtypeStruct((N_NODES, D_H), jnp.float32)] * 2,
    )(Si, ci, Di, Su, cu, Du, Wsi1, Wdi1, bdu1)


def _tc3_body(Sp0, Sp1, cu, Du1, Wout, bout, out):
    h = (Sp0[...] + Sp1[...]) / jnp.maximum(cu[...], 1.0) + Du1[...]
    out[...] = jnp.dot(h, Wout[...], preferred_element_type=jnp.float32) \
        + bout[...]


def _tc3(Sp0, Sp1, cu, Du1, Wout, bout):
    n_out = Wout.shape[1]
    return pl.pallas_call(
        _tc3_body,
        grid=(N_NODES // ROW_BLOCK,),
        in_specs=[_row_spec(D_H), _row_spec(D_H), _row_spec(1), _row_spec(D_H),
                  _full_spec((D_H, n_out)), _full_spec((1, n_out))],
        out_specs=_row_spec(n_out),
        out_shape=jax.ShapeDtypeStruct((N_NODES, n_out), jnp.float32),
    )(Sp0, Sp1, cu, Du1, Wout, bout)


def kernel(x_user, x_item, edge_index_u2i, edge_index_i2u,
           W_in_user, b_in_user, W_in_item, b_in_item,
           W_src_u2i, b_src_u2i, W_dst_u2i, b_dst_u2i,
           W_src_i2u, b_src_i2u, W_dst_i2u, b_dst_i2u,
           W_out, b_out):
    i32 = jnp.int32
    src_u2i = edge_index_u2i[0].astype(i32)
    dst_u2i = edge_index_u2i[1].astype(i32)
    src_i2u = edge_index_i2u[0].astype(i32)
    dst_i2u = edge_index_i2u[1].astype(i32)
    half = src_i2u.shape[0] // 2

    z_rows = jnp.zeros((ROWS_PER_TILE, D_H), jnp.float32)
    z_hist = jnp.zeros((N_PAD,), jnp.float32)

    bDi0 = (b_src_u2i[0] + b_dst_u2i[0])[None]
    bDu0 = (b_src_i2u[0] + b_dst_i2u[0])[None]
    bDu1 = (b_src_i2u[1] + b_dst_i2u[1])[None]

    Au0, Ai0, Di0, Du0 = _tc1(
        x_user, x_item, W_in_user, b_in_user[None], W_in_item, b_in_item[None],
        W_src_u2i[0], W_src_i2u[0], W_dst_u2i[0], W_dst_i2u[0], bDi0, bDu0)

    S1, cnt_flat, _hr = _sc_agg_stage1(Au0, Ai0, src_u2i, dst_u2i, src_i2u,
                                       dst_i2u, z_rows, z_hist)
    cnt = cnt_flat.reshape(N_SC, N_PAD)
    cnt_i = cnt[0, :N_NODES][:, None]
    cnt_u = cnt[1, :N_NODES][:, None]

    Ai1, Du1 = _tc2(S1[0, :N_NODES], cnt_i, Di0, S1[1, :N_NODES], cnt_u, Du0,
                    W_src_i2u[1], W_dst_i2u[1], bDu1)

    S2 = _sc_agg_stage2(Ai1, Ai1, src_i2u[:half], dst_i2u[:half],
                        src_i2u[half:], dst_i2u[half:], z_rows, z_hist)

    return _tc3(S2[0, :N_NODES], S2[1, :N_NODES], cnt_u, Du1,
                W_out, b_out[None])
